# baseline (speedup 1.0000x reference)
import functools

import jax
import jax.numpy as jnp
from jax import lax
from jax.experimental import pallas as pl
from jax.experimental.pallas import tpu as pltpu
from jax.experimental.pallas import tpu_sc as plsc

_ROWS = 16384
_COLS = 1024
_NC, _NS = 2, 16
_NW = _NC * _NS
_RPW = _ROWS // _NW
_CH = 32
_NCH = _RPW // _CH
_KB = 2

_mesh = plsc.VectorSubcoreMesh(core_axis_name="c", subcore_axis_name="s")


@functools.partial(
    pl.kernel,
    mesh=_mesh,
    out_type=jax.ShapeDtypeStruct((_ROWS, _COLS), jnp.float32),
    scratch_types=[
        pltpu.VMEM((_KB, _CH, _COLS), jnp.float32),
        pltpu.SemaphoreType.DMA((_KB,)),
        pltpu.SemaphoreType.DMA((_KB,)),
    ],
)
def _sc_copy(x_hbm, o_hbm, bufs, in_sems, out_sems):
    wid = lax.axis_index("s") * _NC + lax.axis_index("c")
    base = wid * _RPW

    def in_cp(j):
        return pltpu.make_async_copy(
            x_hbm.at[pl.ds(base + j * _CH, _CH)],
            bufs.at[j % _KB],
            in_sems.at[j % _KB],
        )

    def out_cp(j):
        return pltpu.make_async_copy(
            bufs.at[j % _KB],
            o_hbm.at[pl.ds(base + j * _CH, _CH)],
            out_sems.at[j % _KB],
        )

    for j in range(_KB):
        in_cp(j).start()
    for j in range(_NCH):
        in_cp(j).wait()
        out_cp(j).start()
        nxt = j + _KB
        if nxt < _NCH:
            out_cp(j).wait()
            in_cp(nxt).start()
    for j in range(_NCH - _KB, _NCH):
        out_cp(j).wait()


def kernel(x):
    shape = x.shape
    out = _sc_copy(x.reshape(_ROWS, _COLS))
    return out.reshape(shape)


# speedup vs baseline: 1.6103x; 1.6103x over previous
import jax
import jax.numpy as jnp
from jax.experimental import pallas as pl
from jax.experimental.pallas import tpu as pltpu

_SCHED = (4096, 4096, 4096, 4096)
_ARENA_ROWS = 12288
_X_OFF = tuple(sum(_SCHED[:i]) for i in range(len(_SCHED)))
_BUF_OFF = _X_OFF[:3] + (0,)
_BUF_DEPS = {3: (0,)}


def _copy_body(x_ref, o_ref, arena, in_sems, out_sems):
    n = len(_SCHED)

    def in_cp(i):
        return pltpu.make_async_copy(
            x_ref.at[pl.ds(_X_OFF[i], _SCHED[i])],
            arena.at[pl.ds(_BUF_OFF[i], _SCHED[i])],
            in_sems.at[i],
        )

    def out_cp(i):
        return pltpu.make_async_copy(
            arena.at[pl.ds(_BUF_OFF[i], _SCHED[i])],
            o_ref.at[pl.ds(_X_OFF[i], _SCHED[i])],
            out_sems.at[i],
        )

    for j in range(n):
        if j not in _BUF_DEPS:
            in_cp(j).start()
    for i in range(n):
        in_cp(i).wait()
        out_cp(i).start()
        for j, deps in _BUF_DEPS.items():
            if i == max(deps):
                for d in deps:
                    out_cp(d).wait()
                in_cp(j).start()
    for i in range(n):
        if not any(i in deps for deps in _BUF_DEPS.values()):
            out_cp(i).wait()


def kernel(x):
    shape = x.shape
    x2 = x.reshape(-1, shape[-1])
    out = pl.pallas_call(
        _copy_body,
        out_shape=jax.ShapeDtypeStruct(x2.shape, x2.dtype),
        in_specs=[pl.BlockSpec(memory_space=pl.ANY)],
        out_specs=pl.BlockSpec(memory_space=pl.ANY),
        scratch_shapes=[
            pltpu.VMEM((_ARENA_ROWS, x2.shape[1]), x2.dtype),
            pltpu.SemaphoreType.DMA((len(_SCHED),)),
            pltpu.SemaphoreType.DMA((len(_SCHED),)),
        ],
    )(x2)
    return out.reshape(shape)


# speedup vs baseline: 1.6429x; 1.0202x over previous
import jax
import jax.numpy as jnp
from jax.experimental import pallas as pl
from jax.experimental.pallas import tpu as pltpu

_SCHED = (4096, 8192, 4096)
_ARENA_ROWS = 12288
_X_OFF = tuple(sum(_SCHED[:i]) for i in range(len(_SCHED)))
_BUF_OFF = (0, 4096, 0)
_BUF_DEPS = {2: (0,)}


def _copy_body(x_ref, o_ref, arena, in_sems, out_sems):
    n = len(_SCHED)

    def in_cp(i):
        return pltpu.make_async_copy(
            x_ref.at[pl.ds(_X_OFF[i], _SCHED[i])],
            arena.at[pl.ds(_BUF_OFF[i], _SCHED[i])],
            in_sems.at[i],
        )

    def out_cp(i):
        return pltpu.make_async_copy(
            arena.at[pl.ds(_BUF_OFF[i], _SCHED[i])],
            o_ref.at[pl.ds(_X_OFF[i], _SCHED[i])],
            out_sems.at[i],
        )

    for j in range(n):
        if j not in _BUF_DEPS:
            in_cp(j).start()
    for i in range(n):
        in_cp(i).wait()
        out_cp(i).start()
        for j, deps in _BUF_DEPS.items():
            if i == max(deps):
                for d in deps:
                    out_cp(d).wait()
                in_cp(j).start()
    for i in range(n):
        if not any(i in deps for deps in _BUF_DEPS.values()):
            out_cp(i).wait()


def kernel(x):
    shape = x.shape
    x2 = x.reshape(-1, shape[-1])
    out = pl.pallas_call(
        _copy_body,
        out_shape=jax.ShapeDtypeStruct(x2.shape, x2.dtype),
        in_specs=[pl.BlockSpec(memory_space=pl.ANY)],
        out_specs=pl.BlockSpec(memory_space=pl.ANY),
        scratch_shapes=[
            pltpu.VMEM((_ARENA_ROWS, x2.shape[1]), x2.dtype),
            pltpu.SemaphoreType.DMA((len(_SCHED),)),
            pltpu.SemaphoreType.DMA((len(_SCHED),)),
        ],
    )(x2)
    return out.reshape(shape)


# speedup vs baseline: 1.6693x; 1.0161x over previous
import jax
import jax.numpy as jnp
from jax.experimental import pallas as pl
from jax.experimental.pallas import tpu as pltpu

_SCHED = (2048, 12288, 2048)
_ARENA_ROWS = 14336
_X_OFF = tuple(sum(_SCHED[:i]) for i in range(len(_SCHED)))
_BUF_OFF = (0, 2048, 0)
_BUF_DEPS = {2: (0,)}


def _copy_body(x_ref, o_ref, arena, in_sems, out_sems):
    n = len(_SCHED)

    def in_cp(i):
        return pltpu.make_async_copy(
            x_ref.at[pl.ds(_X_OFF[i], _SCHED[i])],
            arena.at[pl.ds(_BUF_OFF[i], _SCHED[i])],
            in_sems.at[i],
        )

    def out_cp(i):
        return pltpu.make_async_copy(
            arena.at[pl.ds(_BUF_OFF[i], _SCHED[i])],
            o_ref.at[pl.ds(_X_OFF[i], _SCHED[i])],
            out_sems.at[i],
        )

    for j in range(n):
        if j not in _BUF_DEPS:
            in_cp(j).start()
    for i in range(n):
        in_cp(i).wait()
        out_cp(i).start()
        for j, deps in _BUF_DEPS.items():
            if i == max(deps):
                for d in deps:
                    out_cp(d).wait()
                in_cp(j).start()
    for i in range(n):
        if not any(i in deps for deps in _BUF_DEPS.values()):
            out_cp(i).wait()


def kernel(x):
    shape = x.shape
    x2 = x.reshape(-1, shape[-1])
    out = pl.pallas_call(
        _copy_body,
        out_shape=jax.ShapeDtypeStruct(x2.shape, x2.dtype),
        in_specs=[pl.BlockSpec(memory_space=pl.ANY)],
        out_specs=pl.BlockSpec(memory_space=pl.ANY),
        scratch_shapes=[
            pltpu.VMEM((_ARENA_ROWS, x2.shape[1]), x2.dtype),
            pltpu.SemaphoreType.DMA((len(_SCHED),)),
            pltpu.SemaphoreType.DMA((len(_SCHED),)),
        ],
    )(x2)
    return out.reshape(shape)
